# baseline (speedup 1.0000x reference)
import functools

import jax
import jax.numpy as jnp
from jax import lax
from jax.experimental import pallas as pl
from jax.experimental.pallas import tpu as pltpu
from jax.experimental.pallas import tpu_sc as plsc

NUM_CORES = 2
NUM_SUBCORES = 16
NUM_WORKERS = NUM_CORES * NUM_SUBCORES
LANES = 16
CS = 8
SETS = 3
UNROLL = 8


def kernel(x, embedding):
    B, S, D = x.shape
    s_per_w = S // NUM_WORKERS
    chunks = s_per_w // CS
    cd = CS * D
    x1 = x.reshape(-1)
    e1 = embedding.reshape(-1)

    mesh = plsc.VectorSubcoreMesh(core_axis_name="c", subcore_axis_name="s")

    scratch = []
    for _ in range(SETS):
        scratch.append(pltpu.VMEM((cd,), jnp.float32))
        for _ in range(B):
            scratch.append(pltpu.VMEM((cd,), jnp.float32))
        scratch.append(pltpu.SemaphoreType.DMA)
        scratch.append(pltpu.SemaphoreType.DMA)

    @functools.partial(
        pl.kernel,
        mesh=mesh,
        out_type=jax.ShapeDtypeStruct((B * S * D,), x.dtype),
        scratch_types=scratch,
    )
    def body(x_hbm, emb_hbm, out_hbm, *scr):
        per = B + 3
        sets = [scr[i * per:(i + 1) * per] for i in range(SETS)]
        wid = lax.axis_index("s") * NUM_CORES + lax.axis_index("c")
        s0 = wid * s_per_w

        def issue_loads(g):
            eb = sets[g % SETS][0]
            xbs = sets[g % SETS][1:1 + B]
            lsem = sets[g % SETS][B + 1]
            hs = [pltpu.async_copy(
                emb_hbm.at[pl.ds((s0 + g * CS) * D, cd)], eb, lsem)]
            for b in range(B):
                hs.append(pltpu.async_copy(
                    x_hbm.at[pl.ds((b * S + s0 + g * CS) * D, cd)],
                    xbs[b], lsem))
            return hs

        def issue_stores(g):
            xbs = sets[g % SETS][1:1 + B]
            ssem = sets[g % SETS][B + 2]
            hs = []
            for b in range(B):
                hs.append(pltpu.async_copy(
                    xbs[b],
                    out_hbm.at[pl.ds((b * S + s0 + g * CS) * D, cd)], ssem))
            return hs

        def compute(g):
            eb = sets[g % SETS][0]
            xbs = sets[g % SETS][1:1 + B]

            @plsc.parallel_loop(0, cd, step=LANES, unroll=UNROLL)
            def iter_body(i):
                sl = pl.ds(i, LANES)
                e = eb[sl]
                for b in range(B):
                    xbs[b][sl] = xbs[b][sl] + e

        load_hs = {}
        store_hs = {}
        for g in range(min(SETS - 1, chunks)):
            load_hs[g] = issue_loads(g)
        for g in range(chunks):
            if g >= 2:
                for h in store_hs.pop(g - 2):
                    h.wait()
            if g + SETS - 1 < chunks:
                load_hs[g + SETS - 1] = issue_loads(g + SETS - 1)
            for h in load_hs.pop(g):
                h.wait()
            compute(g)
            store_hs[g] = issue_stores(g)
        for g in sorted(store_hs):
            for h in store_hs[g]:
                h.wait()

    out = body(x1, e1)
    return out.reshape(B, S, D)


# speedup vs baseline: 1.0217x; 1.0217x over previous
import functools

import jax
import jax.numpy as jnp
from jax import lax
from jax.experimental import pallas as pl
from jax.experimental.pallas import tpu as pltpu
from jax.experimental.pallas import tpu_sc as plsc

NUM_CORES = 2
NUM_SUBCORES = 16
NUM_WORKERS = NUM_CORES * NUM_SUBCORES
LANES = 16
CS = 4
LSLOTS = 4
OSLOTS = 2
UNROLL = 4


def kernel(x, embedding):
    B, S, D = x.shape
    s_per_w = S // NUM_WORKERS
    chunks = s_per_w // CS
    supers = chunks // LSLOTS
    cd = CS * D
    x1 = x.reshape(-1)
    e1 = embedding.reshape(-1)

    mesh = plsc.VectorSubcoreMesh(core_axis_name="c", subcore_axis_name="s")

    scratch = []
    for _ in range(LSLOTS):
        scratch.append(pltpu.VMEM((cd,), jnp.float32))
        for _ in range(B):
            scratch.append(pltpu.VMEM((cd,), jnp.float32))
        scratch.append(pltpu.SemaphoreType.DMA)
    for _ in range(OSLOTS):
        scratch.append(pltpu.VMEM((B * cd,), jnp.float32))
        scratch.append(pltpu.SemaphoreType.DMA)

    @functools.partial(
        pl.kernel,
        mesh=mesh,
        out_type=jax.ShapeDtypeStruct((B * S * D,), x.dtype),
        scratch_types=scratch,
    )
    def body(x_hbm, emb_hbm, out_hbm, *scr):
        lper = B + 2
        lsets = [scr[i * lper:(i + 1) * lper] for i in range(LSLOTS)]
        osets = [scr[LSLOTS * lper + 2 * i:LSLOTS * lper + 2 * i + 2]
                 for i in range(OSLOTS)]
        wid = lax.axis_index("s") * NUM_CORES + lax.axis_index("c")
        s0 = wid * s_per_w

        def issue_loads(ls, gg):
            eb, lsem = lsets[ls][0], lsets[ls][B + 1]
            xbs = lsets[ls][1:1 + B]
            pltpu.async_copy(emb_hbm.at[pl.ds((s0 + gg * CS) * D, cd)],
                             eb, lsem)
            for b in range(B):
                pltpu.async_copy(
                    x_hbm.at[pl.ds((b * S + s0 + gg * CS) * D, cd)],
                    xbs[b], lsem)

        def wait_loads(ls):
            eb, lsem = lsets[ls][0], lsets[ls][B + 1]
            xbs = lsets[ls][1:1 + B]
            pltpu.make_async_copy(emb_hbm.at[pl.ds(0, cd)], eb, lsem).wait()
            for b in range(B):
                pltpu.make_async_copy(x_hbm.at[pl.ds(0, cd)], xbs[b],
                                      lsem).wait()

        def issue_stores(os_, gg):
            ob, ssem = osets[os_]
            for b in range(B):
                pltpu.async_copy(
                    ob.at[pl.ds(b * cd, cd)],
                    out_hbm.at[pl.ds((b * S + s0 + gg * CS) * D, cd)], ssem)

        def wait_stores(os_):
            ob, ssem = osets[os_]
            for b in range(B):
                pltpu.make_async_copy(ob.at[pl.ds(b * cd, cd)],
                                      out_hbm.at[pl.ds(0, cd)], ssem).wait()

        def compute(ls, os_):
            eb = lsets[ls][0]
            xbs = lsets[ls][1:1 + B]
            ob = osets[os_][0]

            @plsc.parallel_loop(0, cd, step=LANES, unroll=UNROLL)
            def iter_body(i):
                sl = pl.ds(i, LANES)
                e = eb[sl]
                for b in range(B):
                    ob[pl.ds(b * cd + i, LANES)] = xbs[b][sl] + e

        for g in range(LSLOTS):
            issue_loads(g, g)

        def chunk_body(gg, i, do_store_wait, do_load_issue):
            ls, os_ = i % LSLOTS, i % OSLOTS
            wait_loads(ls)
            if do_store_wait:
                wait_stores(os_)
            compute(ls, os_)
            if do_load_issue:
                issue_loads(ls, gg + LSLOTS)
            issue_stores(os_, gg)

        for i in range(LSLOTS):
            chunk_body(i, i, i >= OSLOTS, True)

        def super_body(s, c):
            base = s * LSLOTS
            for i in range(LSLOTS):
                chunk_body(base + i, i, True, True)
            return c

        lax.fori_loop(1, supers - 1, super_body, 0)

        base = (supers - 1) * LSLOTS
        for i in range(LSLOTS):
            chunk_body(base + i, i, True, False)

        for i in range(OSLOTS):
            wait_stores(i)

    out = body(x1, e1)
    return out.reshape(B, S, D)


# speedup vs baseline: 2.9733x; 2.9101x over previous
import functools

import jax
import jax.numpy as jnp
from jax import lax
from jax.experimental import pallas as pl
from jax.experimental.pallas import tpu as pltpu
from jax.experimental.pallas import tpu_sc as plsc

NUM_CORES = 2
NUM_SUBCORES = 16
NUM_WORKERS = NUM_CORES * NUM_SUBCORES
LANES = 16
SR = 8
DC = 512
LSLOTS = 4
OSLOTS = 2
UNROLL = 4


def kernel(x, embedding):
    B, S, D = x.shape
    s_per_w = S // NUM_WORKERS
    dh = D // DC
    chunks = (s_per_w // SR) * dh
    supers = chunks // LSLOTS

    mesh = plsc.VectorSubcoreMesh(core_axis_name="c", subcore_axis_name="s")

    scratch = []
    for _ in range(LSLOTS):
        scratch.append(pltpu.VMEM((SR, DC), jnp.float32))
        for _ in range(B):
            scratch.append(pltpu.VMEM((SR, DC), jnp.float32))
        scratch.append(pltpu.SemaphoreType.DMA)
    for _ in range(OSLOTS):
        for _ in range(B):
            scratch.append(pltpu.VMEM((SR, DC), jnp.float32))
        scratch.append(pltpu.SemaphoreType.DMA)

    @functools.partial(
        pl.kernel,
        mesh=mesh,
        out_type=jax.ShapeDtypeStruct((B, S, D), x.dtype),
        scratch_types=scratch,
        compiler_params=pltpu.CompilerParams(use_tc_tiling_on_sc=True),
    )
    def body(x_hbm, emb_hbm, out_hbm, *scr):
        lper = B + 2
        lsets = [scr[i * lper:(i + 1) * lper] for i in range(LSLOTS)]
        obase = LSLOTS * lper
        osets = [scr[obase + i * (B + 1):obase + (i + 1) * (B + 1)]
                 for i in range(OSLOTS)]
        wid = lax.axis_index("s") * NUM_CORES + lax.axis_index("c")
        s0 = wid * s_per_w

        def rowcol(gg):
            return s0 + (gg // dh) * SR, (gg % dh) * DC

        def issue_loads(ls, gg):
            eb, lsem = lsets[ls][0], lsets[ls][B + 1]
            xbs = lsets[ls][1:1 + B]
            sr, dc = rowcol(gg)
            pltpu.async_copy(
                emb_hbm.at[pl.ds(sr, SR), pl.ds(dc, DC)], eb, lsem)
            for b in range(B):
                pltpu.async_copy(
                    x_hbm.at[b, pl.ds(sr, SR), pl.ds(dc, DC)], xbs[b], lsem)

        def wait_loads(ls):
            eb, lsem = lsets[ls][0], lsets[ls][B + 1]
            xbs = lsets[ls][1:1 + B]
            dummy = emb_hbm.at[pl.ds(0, SR), pl.ds(0, DC)]
            pltpu.make_async_copy(dummy, eb, lsem).wait()
            for b in range(B):
                pltpu.make_async_copy(dummy, xbs[b], lsem).wait()

        def issue_stores(os_, gg):
            obs, ssem = osets[os_][:B], osets[os_][B]
            sr, dc = rowcol(gg)
            for b in range(B):
                pltpu.async_copy(
                    obs[b], out_hbm.at[b, pl.ds(sr, SR), pl.ds(dc, DC)],
                    ssem)

        def wait_stores(os_):
            obs, ssem = osets[os_][:B], osets[os_][B]
            dummy = out_hbm.at[0, pl.ds(0, SR), pl.ds(0, DC)]
            for b in range(B):
                pltpu.make_async_copy(obs[b], dummy, ssem).wait()

        def compute(ls, os_):
            eb = lsets[ls][0]
            xbs = lsets[ls][1:1 + B]
            obs = osets[os_][:B]
            for r in range(SR):
                @plsc.parallel_loop(0, DC, step=LANES, unroll=UNROLL)
                def iter_body(i):
                    sl = pl.ds(i, LANES)
                    e = eb[r, sl]
                    for b in range(B):
                        obs[b][r, sl] = xbs[b][r, sl] + e

        def chunk_body(gg, i, do_store_wait, do_load_issue):
            ls, os_ = i % LSLOTS, i % OSLOTS
            wait_loads(ls)
            if do_store_wait:
                wait_stores(os_)
            compute(ls, os_)
            if do_load_issue:
                issue_loads(ls, gg + LSLOTS)
            issue_stores(os_, gg)

        for g in range(LSLOTS):
            issue_loads(g, g)

        for i in range(LSLOTS):
            chunk_body(i, i, i >= OSLOTS, True)

        def super_body(s, c):
            base = s * LSLOTS
            for i in range(LSLOTS):
                chunk_body(base + i, i, True, True)
            return c

        lax.fori_loop(1, supers - 1, super_body, 0)

        base = (supers - 1) * LSLOTS
        for i in range(LSLOTS):
            chunk_body(base + i, i, True, False)

        for i in range(OSLOTS):
            wait_stores(i)

    return body(x, embedding)


# speedup vs baseline: 3.1255x; 1.0512x over previous
import functools

import jax
import jax.numpy as jnp
from jax import lax
from jax.experimental import pallas as pl
from jax.experimental.pallas import tpu as pltpu
from jax.experimental.pallas import tpu_sc as plsc

NUM_CORES = 2
NUM_SUBCORES = 16
NUM_WORKERS = NUM_CORES * NUM_SUBCORES
LANES = 16
SR = 8
DC = 512
LSLOTS = 2
OSLOTS = 2
UNROLL = 4


def kernel(x, embedding):
    B, S, D = x.shape
    s_per_w = S // NUM_WORKERS
    dh = D // DC
    chunks = (s_per_w // SR) * dh
    supers = chunks // LSLOTS

    mesh = plsc.VectorSubcoreMesh(core_axis_name="c", subcore_axis_name="s")

    scratch = []
    for _ in range(LSLOTS):
        scratch.append(pltpu.VMEM((SR, DC), jnp.float32))
        scratch.append(pltpu.VMEM((B, SR, DC), jnp.float32))
        scratch.append(pltpu.SemaphoreType.DMA)
    for _ in range(OSLOTS):
        scratch.append(pltpu.VMEM((B, SR, DC), jnp.float32))
        scratch.append(pltpu.SemaphoreType.DMA)

    @functools.partial(
        pl.kernel,
        mesh=mesh,
        out_type=jax.ShapeDtypeStruct((B, S, D), x.dtype),
        scratch_types=scratch,
        compiler_params=pltpu.CompilerParams(use_tc_tiling_on_sc=True),
    )
    def body(x_hbm, emb_hbm, out_hbm, *scr):
        lsets = [scr[3 * i:3 * i + 3] for i in range(LSLOTS)]
        obase = 3 * LSLOTS
        osets = [scr[obase + 2 * i:obase + 2 * i + 2] for i in range(OSLOTS)]
        wid = lax.axis_index("s") * NUM_CORES + lax.axis_index("c")
        s0 = wid * s_per_w

        def rowcol(gg):
            return s0 + (gg // dh) * SR, (gg % dh) * DC

        def issue_loads(ls, gg):
            eb, xb, lsem = lsets[ls]
            sr, dc = rowcol(gg)
            pltpu.async_copy(
                emb_hbm.at[pl.ds(sr, SR), pl.ds(dc, DC)], eb, lsem)
            pltpu.async_copy(
                x_hbm.at[pl.ds(0, B), pl.ds(sr, SR), pl.ds(dc, DC)],
                xb, lsem)

        def wait_loads(ls):
            eb, xb, lsem = lsets[ls]
            pltpu.make_async_copy(
                emb_hbm.at[pl.ds(0, SR), pl.ds(0, DC)], eb, lsem).wait()
            pltpu.make_async_copy(
                x_hbm.at[pl.ds(0, B), pl.ds(0, SR), pl.ds(0, DC)],
                xb, lsem).wait()

        def issue_stores(os_, gg):
            ob, ssem = osets[os_]
            sr, dc = rowcol(gg)
            pltpu.async_copy(
                ob, out_hbm.at[pl.ds(0, B), pl.ds(sr, SR), pl.ds(dc, DC)],
                ssem)

        def wait_stores(os_):
            ob, ssem = osets[os_]
            pltpu.make_async_copy(
                ob, out_hbm.at[pl.ds(0, B), pl.ds(0, SR), pl.ds(0, DC)],
                ssem).wait()

        def compute(ls, os_):
            eb, xb, _ = lsets[ls]
            ob = osets[os_][0]

            @plsc.parallel_loop(0, SR * DC, step=LANES, unroll=UNROLL)
            def iter_body(i):
                r = i // DC
                sl = pl.ds(i % DC, LANES)
                e = eb[r, sl]
                for b in range(B):
                    ob[b, r, sl] = xb[b, r, sl] + e

        def chunk_body(gg, i, do_store_wait, do_load_issue):
            ls, os_ = i % LSLOTS, i % OSLOTS
            wait_loads(ls)
            if do_store_wait:
                wait_stores(os_)
            compute(ls, os_)
            if do_load_issue:
                issue_loads(ls, gg + LSLOTS)
            issue_stores(os_, gg)

        for g in range(LSLOTS):
            issue_loads(g, g)

        for i in range(LSLOTS):
            chunk_body(i, i, i >= OSLOTS, True)

        def super_body(s, c):
            base = s * LSLOTS
            for i in range(LSLOTS):
                chunk_body(base + i, i, True, True)
            return c

        lax.fori_loop(1, supers - 1, super_body, 0)

        base = (supers - 1) * LSLOTS
        for i in range(LSLOTS):
            chunk_body(base + i, i, True, False)

        for i in range(OSLOTS):
            wait_stores(i)

    return body(x, embedding)
